# baseline (device time: 236243 ns/iter reference)
import jax
import jax.numpy as jnp
from jax import lax
from jax.experimental import pallas as pl
from jax.experimental.pallas import tpu as pltpu

C = 256
LAG = 2
LEAD = 2


def kernel(partial, resid, gamma):
    _, M, D = partial.shape
    H = M // 2
    K = H // C

    def body(partial_ref, resid_ref, gamma_ref, out_ref,
             pa, rs, ob, xsend, xrecv, yrecv,
             pa_sems, rs_sems, ob_sems, yc_sems,
             xsend_sems, xrecv_sems, ysend_sems, yrecv_sems):
        my_x = lax.axis_index("x")
        my_y = lax.axis_index("y")
        my_z = lax.axis_index("z")
        xnbr = (1 - my_x, my_y, my_z)
        ynbr = (my_x, 1 - my_y, my_z)

        row0 = my_y * H
        orow0 = (1 - my_y) * H

        barrier_sem = pltpu.get_barrier_semaphore()
        for nbr in (xnbr, ynbr):
            pl.semaphore_signal(barrier_sem, inc=1, device_id=nbr,
                                device_id_type=pl.DeviceIdType.MESH)
        pl.semaphore_wait(barrier_sem, 2)

        def pa_dma(k):
            return pltpu.make_async_copy(
                partial_ref.at[0, pl.ds(row0 + k * C, C), :],
                pa.at[k % 2], pa_sems.at[k % 2])

        def rs_dma(k):
            return pltpu.make_async_copy(
                resid_ref.at[pl.ds(row0 + k * C, C), :],
                rs.at[k % 2], rs_sems.at[k % 2])

        def ob_dma(k):
            return pltpu.make_async_copy(
                ob.at[k % 2], out_ref.at[pl.ds(row0 + k * C, C), :],
                ob_sems.at[k % 2])

        def yc_dma(j):
            return pltpu.make_async_copy(
                yrecv.at[j], out_ref.at[pl.ds(orow0 + j * C, C), :],
                yc_sems.at[j % 2])

        def rdma_x(k):
            return pltpu.make_async_remote_copy(
                src_ref=xsend.at[k % 3], dst_ref=xrecv.at[k],
                send_sem=xsend_sems.at[k % 3], recv_sem=xrecv_sems.at[k],
                device_id=xnbr, device_id_type=pl.DeviceIdType.MESH)

        def rdma_y(k):
            return pltpu.make_async_remote_copy(
                src_ref=ob.at[k % 2], dst_ref=yrecv.at[k],
                send_sem=ysend_sems.at[k % 2], recv_sem=yrecv_sems.at[k],
                device_id=ynbr, device_id_type=pl.DeviceIdType.MESH)

        def consume_y(j):
            rdma_y(j).wait_recv()
            if j >= 2:
                yc_dma(j - 2).wait()
            yc_dma(j).start()

        gamma_row = gamma_ref[...][None, :]

        def stage1(k):
            if k >= 3:
                rdma_x(k - 3).wait_send()
            pa_dma(k).wait()
            xsend[k % 3] = pa[k % 2].astype(jnp.bfloat16)
            if k + 2 < K:
                pa_dma(k + 2).start()
            rdma_x(k).start()

        for k in range(min(2, K)):
            pa_dma(k).start()
        rs_dma(0).start()
        for k in range(min(LEAD, K)):
            stage1(k)

        for k in range(K):
            if k + LEAD < K:
                stage1(k + LEAD)

            if k + 1 < K:
                rs_dma(k + 1).start()
            rdma_x(k).wait_recv()
            rs_dma(k).wait()
            y = (xsend[k % 3].astype(jnp.float32)
                 + xrecv[k].astype(jnp.float32) + rs[k % 2])
            ms = jnp.mean(y * y, axis=-1, keepdims=True)
            if k >= 2:
                ob_dma(k - 2).wait()
                rdma_y(k - 2).wait_send()
            ob[k % 2] = (y * lax.rsqrt(ms + 1e-6) * gamma_row
                         ).astype(jnp.bfloat16)
            ob_dma(k).start()
            rdma_y(k).start()

            if k >= LAG:
                consume_y(k - LAG)

        for j in range(K - LAG, K):
            consume_y(j)
        for k in range(K - 3, K):
            rdma_x(k).wait_send()
        for k in (K - 2, K - 1):
            rdma_y(k).wait_send()
            ob_dma(k).wait()
            yc_dma(k).wait()

    out_shape = jax.ShapeDtypeStruct((M, D), jnp.bfloat16)
    return pl.pallas_call(
        body,
        out_shape=out_shape,
        in_specs=[
            pl.BlockSpec(memory_space=pl.ANY),
            pl.BlockSpec(memory_space=pl.ANY),
            pl.BlockSpec(memory_space=pltpu.VMEM),
        ],
        out_specs=pl.BlockSpec(memory_space=pl.ANY),
        scratch_shapes=[
            pltpu.VMEM((2, C, D), jnp.float32),
            pltpu.VMEM((2, C, D), jnp.float32),
            pltpu.VMEM((2, C, D), jnp.bfloat16),
            pltpu.VMEM((3, C, D), jnp.bfloat16),
            pltpu.VMEM((M // 2 // C, C, D), jnp.bfloat16),
            pltpu.VMEM((M // 2 // C, C, D), jnp.bfloat16),
            pltpu.SemaphoreType.DMA((2,)),
            pltpu.SemaphoreType.DMA((2,)),
            pltpu.SemaphoreType.DMA((2,)),
            pltpu.SemaphoreType.DMA((2,)),
            pltpu.SemaphoreType.DMA((3,)),
            pltpu.SemaphoreType.DMA((M // 2 // C,)),
            pltpu.SemaphoreType.DMA((2,)),
            pltpu.SemaphoreType.DMA((M // 2 // C,)),
        ],
        compiler_params=pltpu.CompilerParams(
            collective_id=0, vmem_limit_bytes=100 * 1024 * 1024),
    )(partial, resid, gamma)


# device time: 222628 ns/iter; 1.0612x vs baseline; 1.0612x over previous
import jax
import jax.numpy as jnp
from jax import lax
from jax.experimental import pallas as pl
from jax.experimental.pallas import tpu as pltpu

CMAX = 128
SIZES = [64, 64] + [128] * 14 + [64, 64]
OFFS = [sum(SIZES[:i]) for i in range(len(SIZES))]
LAG = 2
LEAD = 3


def kernel(partial, resid, gamma):
    _, M, D = partial.shape
    H = M // 2
    K = len(SIZES)
    assert sum(SIZES) == H

    def body(partial_ref, resid_ref, gamma_ref, out_ref,
             pa, rs, ob, xsend, xrecv, yrecv,
             pa_sems, rs_sems, ob_sems, yc_sems,
             xsend_sems, xrecv_sems, ysend_sems, yrecv_sems):
        my_x = lax.axis_index("x")
        my_y = lax.axis_index("y")
        my_z = lax.axis_index("z")
        xnbr = (1 - my_x, my_y, my_z)
        ynbr = (my_x, 1 - my_y, my_z)

        row0 = my_y * H
        orow0 = (1 - my_y) * H

        barrier_sem = pltpu.get_barrier_semaphore()
        for nbr in (xnbr, ynbr):
            pl.semaphore_signal(barrier_sem, inc=1, device_id=nbr,
                                device_id_type=pl.DeviceIdType.MESH)
        pl.semaphore_wait(barrier_sem, 2)

        def pa_dma(k):
            return pltpu.make_async_copy(
                partial_ref.at[0, pl.ds(row0 + OFFS[k], SIZES[k]), :],
                pa.at[k % 4, pl.ds(0, SIZES[k]), :], pa_sems.at[k % 4])

        def rs_dma(k):
            return pltpu.make_async_copy(
                resid_ref.at[pl.ds(row0 + OFFS[k], SIZES[k]), :],
                rs.at[k % 2, pl.ds(0, SIZES[k]), :], rs_sems.at[k % 2])

        def ob_dma(k):
            return pltpu.make_async_copy(
                ob.at[k % 2, pl.ds(0, SIZES[k]), :],
                out_ref.at[pl.ds(row0 + OFFS[k], SIZES[k]), :],
                ob_sems.at[k % 2])

        def yc_dma(j):
            return pltpu.make_async_copy(
                yrecv.at[j, pl.ds(0, SIZES[j]), :],
                out_ref.at[pl.ds(orow0 + OFFS[j], SIZES[j]), :],
                yc_sems.at[j % 2])

        def rdma_x(k):
            return pltpu.make_async_remote_copy(
                src_ref=xsend.at[k % 4, pl.ds(0, SIZES[k]), :],
                dst_ref=xrecv.at[k, pl.ds(0, SIZES[k]), :],
                send_sem=xsend_sems.at[k % 4], recv_sem=xrecv_sems.at[k],
                device_id=xnbr, device_id_type=pl.DeviceIdType.MESH)

        def rdma_y(k):
            return pltpu.make_async_remote_copy(
                src_ref=ob.at[k % 2, pl.ds(0, SIZES[k]), :],
                dst_ref=yrecv.at[k, pl.ds(0, SIZES[k]), :],
                send_sem=ysend_sems.at[k % 2], recv_sem=yrecv_sems.at[k],
                device_id=ynbr, device_id_type=pl.DeviceIdType.MESH)

        def consume_y(j):
            rdma_y(j).wait_recv()
            if j >= 2:
                yc_dma(j - 2).wait()
            yc_dma(j).start()

        gamma_row = gamma_ref[...][None, :]

        def stage1(k):
            if k >= 4:
                rdma_x(k - 4).wait_send()
            pa_dma(k).wait()
            s = pl.ds(0, SIZES[k])
            xsend[k % 4, s, :] = pa[k % 4, s, :].astype(jnp.bfloat16)
            rdma_x(k).start()

        for k in range(min(4, K)):
            pa_dma(k).start()
        rs_dma(0).start()
        for k in range(min(LEAD, K)):
            stage1(k)

        for k in range(K):
            if k + LEAD < K:
                stage1(k + LEAD)

            if k + 1 < K:
                rs_dma(k + 1).start()
            rdma_x(k).wait_recv()
            rs_dma(k).wait()
            s = pl.ds(0, SIZES[k])
            y = (pa[k % 4, s, :] + xrecv[k, s, :].astype(jnp.float32)
                 + rs[k % 2, s, :])
            if k + LEAD + 1 < K:
                pa_dma(k + LEAD + 1).start()
            ms = jnp.mean(y * y, axis=-1, keepdims=True)
            if k >= 2:
                ob_dma(k - 2).wait()
                rdma_y(k - 2).wait_send()
            ob[k % 2, s, :] = (y * lax.rsqrt(ms + 1e-6) * gamma_row
                               ).astype(jnp.bfloat16)
            ob_dma(k).start()
            rdma_y(k).start()

            if k >= LAG:
                consume_y(k - LAG)

        for j in range(K - LAG, K):
            consume_y(j)
        for k in range(K - 4, K):
            rdma_x(k).wait_send()
        for k in (K - 2, K - 1):
            rdma_y(k).wait_send()
            ob_dma(k).wait()
            yc_dma(k).wait()

    out_shape = jax.ShapeDtypeStruct((M, D), jnp.bfloat16)
    return pl.pallas_call(
        body,
        out_shape=out_shape,
        in_specs=[
            pl.BlockSpec(memory_space=pl.ANY),
            pl.BlockSpec(memory_space=pl.ANY),
            pl.BlockSpec(memory_space=pltpu.VMEM),
        ],
        out_specs=pl.BlockSpec(memory_space=pl.ANY),
        scratch_shapes=[
            pltpu.VMEM((4, CMAX, D), jnp.float32),
            pltpu.VMEM((2, CMAX, D), jnp.float32),
            pltpu.VMEM((2, CMAX, D), jnp.bfloat16),
            pltpu.VMEM((4, CMAX, D), jnp.bfloat16),
            pltpu.VMEM((len(SIZES), CMAX, D), jnp.bfloat16),
            pltpu.VMEM((len(SIZES), CMAX, D), jnp.bfloat16),
            pltpu.SemaphoreType.DMA((4,)),
            pltpu.SemaphoreType.DMA((2,)),
            pltpu.SemaphoreType.DMA((2,)),
            pltpu.SemaphoreType.DMA((2,)),
            pltpu.SemaphoreType.DMA((4,)),
            pltpu.SemaphoreType.DMA((len(SIZES),)),
            pltpu.SemaphoreType.DMA((2,)),
            pltpu.SemaphoreType.DMA((len(SIZES),)),
        ],
        compiler_params=pltpu.CompilerParams(
            collective_id=0, vmem_limit_bytes=100 * 1024 * 1024),
    )(partial, resid, gamma)
